# baseline (speedup 1.0000x reference)
import jax
import jax.numpy as jnp
from jax import lax
from jax.experimental import pallas as pl
from jax.experimental.pallas import tpu as pltpu
from jax.experimental.pallas import tpu_sc as plsc

N = 10000
E = 320000
D = 128
CW = 16
NC = 2
NS = 16
EPT = E // NS
CH = 80
NCHUNK = EPT // CH
SB = 25
NSB = NCHUNK // SB
NBUF = 10
NP = N
RPT = NP // NS


def _sc_body(xu_hbm, xb_hbm, src_hbm, dst_hbm, zf_hbm, zc_hbm, ones_hbm,
             feat_hbm, cnt_hbm,
             src_v, dst_v, rows_v, ones_v, feat_sh, cnt_sh, sem, semi):
    c = lax.axis_index("c")
    s = lax.axis_index("s")

    pltpu.sync_copy(ones_hbm, ones_v)
    pltpu.sync_copy(zf_hbm, feat_sh.at[pl.ds(s * RPT, RPT)])
    pltpu.sync_copy(zc_hbm, cnt_sh.at[pl.ds(s * RPT, RPT)])
    plsc.subcore_barrier()

    def run_dir(x_hbm):
        pltpu.async_copy(src_hbm.at[c, s, 0], src_v.at[0], semi)
        pltpu.async_copy(dst_hbm.at[c, s, 0], dst_v.at[0], semi)
        pltpu.make_async_copy(src_hbm.at[c, s, 0], src_v.at[0], semi).wait()
        pltpu.make_async_copy(dst_hbm.at[c, s, 0], dst_v.at[0], semi).wait()
        pltpu.async_copy(src_hbm.at[c, s, 1], src_v.at[1], semi)
        pltpu.async_copy(dst_hbm.at[c, s, 1], dst_v.at[1], semi)
        for i in range(NBUF - 1):
            pltpu.async_copy(x_hbm.at[src_v.at[0, i]], rows_v.at[i], sem)

        def chunk(j, _):
            sb = lax.div(j, SB)
            off = lax.rem(j, SB)
            p = lax.rem(sb, 2)
            b = lax.rem(j, NBUF)

            pltpu.make_async_copy(x_hbm.at[src_v.at[p, off]], rows_v.at[b],
                                  sem).wait()

            @pl.when((off == 0) & (j > 0) & (sb + 1 < NSB))
            def _():
                pltpu.async_copy(src_hbm.at[c, s, sb + 1],
                                 src_v.at[1 - p], semi)
                pltpu.async_copy(dst_hbm.at[c, s, sb + 1],
                                 dst_v.at[1 - p], semi)

            t = j + NBUF - 1

            @pl.when(t < NCHUNK)
            def _():
                sbt = lax.div(t, SB)
                offt = lax.rem(t, SB)
                pt = lax.rem(sbt, 2)

                @pl.when(offt == 0)
                def _():
                    pltpu.make_async_copy(src_hbm.at[c, s, sbt],
                                          src_v.at[pt], semi).wait()
                    pltpu.make_async_copy(dst_hbm.at[c, s, sbt],
                                          dst_v.at[pt], semi).wait()

                pltpu.async_copy(x_hbm.at[src_v.at[pt, offt]],
                                 rows_v.at[lax.rem(t, NBUF)], sem)

            pltpu.sync_copy(rows_v.at[b], feat_sh.at[dst_v.at[p, off]],
                            add=True)
            pltpu.sync_copy(ones_v, cnt_sh.at[dst_v.at[p, off]], add=True)
            return ()

        lax.fori_loop(0, NCHUNK, chunk, (), unroll=False)

    @pl.when(c == 0)
    def _():
        run_dir(xu_hbm)

    @pl.when(c == 1)
    def _():
        run_dir(xb_hbm)

    plsc.subcore_barrier()
    pltpu.sync_copy(feat_sh.at[pl.ds(s * RPT, RPT)],
                    feat_hbm.at[c, pl.ds(s * RPT, RPT)])
    pltpu.sync_copy(cnt_sh.at[pl.ds(s * RPT, RPT)],
                    cnt_hbm.at[c, pl.ds(s * RPT, RPT)])


_sc_call = pl.kernel(
    _sc_body,
    out_type=(jax.ShapeDtypeStruct((NC, NP, D), jnp.bfloat16),
              jax.ShapeDtypeStruct((NC, NP, CW), jnp.float32)),
    mesh=plsc.VectorSubcoreMesh(core_axis_name="c", subcore_axis_name="s"),
    scratch_types=[
        pltpu.VMEM((2, SB, CH), jnp.int32),
        pltpu.VMEM((2, SB, CH), jnp.int32),
        pltpu.VMEM((NBUF, CH, D), jnp.bfloat16),
        pltpu.VMEM((CH, CW), jnp.float32),
        pltpu.VMEM_SHARED((NP, D), jnp.bfloat16),
        pltpu.VMEM_SHARED((NP, CW), jnp.float32),
        pltpu.SemaphoreType.DMA,
        pltpu.SemaphoreType.DMA,
    ],
    compiler_params=pltpu.CompilerParams(use_tc_tiling_on_sc=False),
)


_RB = 2000


def _tc_body(sums_ref, cnt_ref, x_ref, wl_ref, wr_ref, b_ref, o_ref):
    cnt = jnp.max(cnt_ref[0], axis=1, keepdims=True)
    mean = sums_ref[0].astype(jnp.float32) / jnp.maximum(cnt, 1.0)
    r = (jnp.dot(mean, wl_ref[...], preferred_element_type=jnp.float32)
         + b_ref[...]
         + jnp.dot(x_ref[...], wr_ref[...], preferred_element_type=jnp.float32))
    o_ref[...] = jnp.maximum(r, 0.0)


def _make_tc_call(d):
    return pl.pallas_call(
        _tc_body,
        grid=(N // _RB,),
        in_specs=[
            pl.BlockSpec((1, _RB, D), lambda i: (d, i, 0)),
            pl.BlockSpec((1, _RB, CW), lambda i: (d, i, 0)),
            pl.BlockSpec((_RB, D), lambda i: (i, 0)),
            pl.BlockSpec((D, D), lambda i: (0, 0)),
            pl.BlockSpec((D, D), lambda i: (0, 0)),
            pl.BlockSpec((1, D), lambda i: (0, 0)),
        ],
        out_specs=pl.BlockSpec((_RB, D), lambda i: (i, 0)),
        out_shape=jax.ShapeDtypeStruct((N, D), jnp.float32),
    )


_tc_calls = (_make_tc_call(0), _make_tc_call(1))


def kernel(x_user, x_book, edge_index_user_to_book, edge_index_book_to_user,
           W_l, b_l, W_r):
    ei_ub = edge_index_user_to_book.astype(jnp.int32)
    ei_bu = edge_index_book_to_user.astype(jnp.int32)

    src = jnp.stack([ei_ub[0], ei_bu[0]]).reshape(NC, NS, NSB, SB, CH)
    dst = jnp.stack([ei_ub[1], ei_bu[1]]).reshape(NC, NS, NSB, SB, CH)
    zf = jnp.zeros((RPT, D), jnp.bfloat16)
    zc = jnp.zeros((RPT, CW), jnp.float32)
    ones = jnp.ones((CH, CW), jnp.float32)

    xu16 = x_user.astype(jnp.bfloat16)
    xb16 = x_book.astype(jnp.bfloat16)
    sums, cnt = _sc_call(xu16, xb16, src, dst, zf, zc, ones)

    WlT, WrT, b2 = W_l.T, W_r.T, b_l.reshape(1, D)
    out_book = _tc_calls[0](sums, cnt, x_book, WlT, WrT, b2)
    out_user = _tc_calls[1](sums, cnt, x_user, WlT, WrT, b2)
    return (out_book, out_user)


# speedup vs baseline: 1.0008x; 1.0008x over previous
import jax
import jax.numpy as jnp
from jax import lax
from jax.experimental import pallas as pl
from jax.experimental.pallas import tpu as pltpu
from jax.experimental.pallas import tpu_sc as plsc

N = 10000
E = 320000
D = 128
CW = 16
NC = 2
NS = 16
EPT = E // NS
CH = 80
NCHUNK = EPT // CH
SB = 25
NSB = NCHUNK // SB
NBUF = 6
NP = N
RPT = NP // NS


def _sc_body(xu_hbm, xb_hbm, src_hbm, dst_hbm, zf_hbm, zc_hbm, ones_hbm,
             feat_hbm, cnt_hbm,
             src_v, dst_v, rows_v, ones_v, feat_sh, cnt_sh, sem, semi):
    c = lax.axis_index("c")
    s = lax.axis_index("s")

    pltpu.sync_copy(ones_hbm, ones_v)
    pltpu.sync_copy(zf_hbm, feat_sh.at[pl.ds(s * RPT, RPT)])
    pltpu.sync_copy(zc_hbm, cnt_sh.at[pl.ds(s * RPT, RPT)])
    plsc.subcore_barrier()

    def run_dir(x_hbm):
        pltpu.async_copy(src_hbm.at[c, s, 0], src_v.at[0], semi)
        pltpu.async_copy(dst_hbm.at[c, s, 0], dst_v.at[0], semi)
        pltpu.make_async_copy(src_hbm.at[c, s, 0], src_v.at[0], semi).wait()
        pltpu.make_async_copy(dst_hbm.at[c, s, 0], dst_v.at[0], semi).wait()
        pltpu.async_copy(src_hbm.at[c, s, 1], src_v.at[1], semi)
        pltpu.async_copy(dst_hbm.at[c, s, 1], dst_v.at[1], semi)
        for i in range(NBUF - 1):
            pltpu.async_copy(x_hbm.at[src_v.at[0, i]], rows_v.at[i], sem)

        def chunk(j, _):
            sb = lax.div(j, SB)
            off = lax.rem(j, SB)
            p = lax.rem(sb, 2)
            b = lax.rem(j, NBUF)

            pltpu.make_async_copy(x_hbm.at[src_v.at[p, off]], rows_v.at[b],
                                  sem).wait()

            @pl.when((off == 0) & (j > 0) & (sb + 1 < NSB))
            def _():
                pltpu.async_copy(src_hbm.at[c, s, sb + 1],
                                 src_v.at[1 - p], semi)
                pltpu.async_copy(dst_hbm.at[c, s, sb + 1],
                                 dst_v.at[1 - p], semi)

            t = j + NBUF - 1

            @pl.when(t < NCHUNK)
            def _():
                sbt = lax.div(t, SB)
                offt = lax.rem(t, SB)
                pt = lax.rem(sbt, 2)

                @pl.when(offt == 0)
                def _():
                    pltpu.make_async_copy(src_hbm.at[c, s, sbt],
                                          src_v.at[pt], semi).wait()
                    pltpu.make_async_copy(dst_hbm.at[c, s, sbt],
                                          dst_v.at[pt], semi).wait()

                pltpu.async_copy(x_hbm.at[src_v.at[pt, offt]],
                                 rows_v.at[lax.rem(t, NBUF)], sem)

            pltpu.sync_copy(rows_v.at[b], feat_sh.at[dst_v.at[p, off]],
                            add=True)
            pltpu.sync_copy(ones_v, cnt_sh.at[dst_v.at[p, off]], add=True)
            return ()

        lax.fori_loop(0, NCHUNK, chunk, (), unroll=False)

    @pl.when(c == 0)
    def _():
        run_dir(xu_hbm)

    @pl.when(c == 1)
    def _():
        run_dir(xb_hbm)

    plsc.subcore_barrier()
    pltpu.sync_copy(feat_sh.at[pl.ds(s * RPT, RPT)],
                    feat_hbm.at[c, pl.ds(s * RPT, RPT)])
    pltpu.sync_copy(cnt_sh.at[pl.ds(s * RPT, RPT)],
                    cnt_hbm.at[c, pl.ds(s * RPT, RPT)])


_sc_call = pl.kernel(
    _sc_body,
    out_type=(jax.ShapeDtypeStruct((NC, NP, D), jnp.bfloat16),
              jax.ShapeDtypeStruct((NC, NP, CW), jnp.float32)),
    mesh=plsc.VectorSubcoreMesh(core_axis_name="c", subcore_axis_name="s"),
    scratch_types=[
        pltpu.VMEM((2, SB, CH), jnp.int32),
        pltpu.VMEM((2, SB, CH), jnp.int32),
        pltpu.VMEM((NBUF, CH, D), jnp.bfloat16),
        pltpu.VMEM((CH, CW), jnp.float32),
        pltpu.VMEM_SHARED((NP, D), jnp.bfloat16),
        pltpu.VMEM_SHARED((NP, CW), jnp.float32),
        pltpu.SemaphoreType.DMA,
        pltpu.SemaphoreType.DMA,
    ],
    compiler_params=pltpu.CompilerParams(use_tc_tiling_on_sc=False),
)


_RB = 2000


def _tc_body(sums_ref, cnt_ref, x_ref, wl_ref, wr_ref, b_ref, o_ref):
    cnt = jnp.max(cnt_ref[0], axis=1, keepdims=True)
    mean = sums_ref[0].astype(jnp.float32) / jnp.maximum(cnt, 1.0)
    r = (jnp.dot(mean, wl_ref[...], preferred_element_type=jnp.float32)
         + b_ref[...]
         + jnp.dot(x_ref[...], wr_ref[...], preferred_element_type=jnp.float32))
    o_ref[...] = jnp.maximum(r, 0.0)


def _make_tc_call(d):
    return pl.pallas_call(
        _tc_body,
        grid=(N // _RB,),
        in_specs=[
            pl.BlockSpec((1, _RB, D), lambda i: (d, i, 0)),
            pl.BlockSpec((1, _RB, CW), lambda i: (d, i, 0)),
            pl.BlockSpec((_RB, D), lambda i: (i, 0)),
            pl.BlockSpec((D, D), lambda i: (0, 0)),
            pl.BlockSpec((D, D), lambda i: (0, 0)),
            pl.BlockSpec((1, D), lambda i: (0, 0)),
        ],
        out_specs=pl.BlockSpec((_RB, D), lambda i: (i, 0)),
        out_shape=jax.ShapeDtypeStruct((N, D), jnp.float32),
    )


_tc_calls = (_make_tc_call(0), _make_tc_call(1))


def kernel(x_user, x_book, edge_index_user_to_book, edge_index_book_to_user,
           W_l, b_l, W_r):
    ei_ub = edge_index_user_to_book.astype(jnp.int32)
    ei_bu = edge_index_book_to_user.astype(jnp.int32)

    src = jnp.stack([ei_ub[0], ei_bu[0]]).reshape(NC, NS, NSB, SB, CH)
    dst = jnp.stack([ei_ub[1], ei_bu[1]]).reshape(NC, NS, NSB, SB, CH)
    zf = jnp.zeros((RPT, D), jnp.bfloat16)
    zc = jnp.zeros((RPT, CW), jnp.float32)
    ones = jnp.ones((CH, CW), jnp.float32)

    xu16 = x_user.astype(jnp.bfloat16)
    xb16 = x_book.astype(jnp.bfloat16)
    sums, cnt = _sc_call(xu16, xb16, src, dst, zf, zc, ones)

    WlT, WrT, b2 = W_l.T, W_r.T, b_l.reshape(1, D)
    out_book = _tc_calls[0](sums, cnt, x_book, WlT, WrT, b2)
    out_user = _tc_calls[1](sums, cnt, x_user, WlT, WrT, b2)
    return (out_book, out_user)
